# Initial kernel scaffold; baseline (speedup 1.0000x reference)
import jax
import jax.numpy as jnp
from jax.experimental import pallas as pl


def kernel(xs, ts, ews, edge_attr_vote, params, edge_indices):
    raise NotImplementedError("write your pallas kernel here")


# speedup vs baseline: 1.2557x; 1.2557x over previous
import functools

import numpy as np
import jax
import jax.numpy as jnp
from jax import lax
from jax.experimental import pallas as pl
from jax.experimental.pallas import tpu as pltpu
from jax.experimental.pallas import tpu_sc as plsc

_NT = ['legislator_term', 'bill', 'bill_version', 'party', 'committee',
       'subject', 'donor']
_TS = ('legislator_term', 'bill', 'bill_version')
_ET = [('legislator_term', 'voted_on', 'bill_version'),
       ('bill', 'has_version', 'bill_version'),
       ('bill', 'about', 'subject'),
       ('donor', 'donated_to', 'legislator_term')]
_NN = {'legislator_term': 10000, 'bill': 20000, 'bill_version': 50000,
       'party': 10, 'committee': 200, 'subject': 1000, 'donor': 20000}
_D = 192
_DK = 48
_BN = 256
_EB = 128
_NW = 32

_BM_NP = (np.arange(_D)[:, None] // _DK == np.arange(_D)[None, :] // _DK
          ).astype(np.float32)


def _ru(n, m):
    return (n + m - 1) // m * m


def _gelu(x):
    return 0.5 * x * (1.0 + lax.erf(x * np.float32(1.0 / np.sqrt(2.0))))


def _ln(x, mu, var, scale, bias):
    return (x - mu) * lax.rsqrt(var + 1e-5) * scale + bias


def _proj_plain_body(x_ref, s_ref, b_ref, w_ref, o_ref):
    x = x_ref[...]
    mu = jnp.mean(x, axis=-1, keepdims=True)
    var = jnp.mean((x - mu) ** 2, axis=-1, keepdims=True)
    y = _ln(x, mu, var, s_ref[...], b_ref[...])
    o_ref[...] = _gelu(jnp.dot(y, w_ref[...], preferred_element_type=jnp.float32))


def _proj_ts_body(x_ref, t_ref, wf_ref, bf_ref, sx_ref, bx_ref, st_ref,
                  bt_ref, wx_ref, wt_ref, o_ref):
    x = x_ref[...]
    t = t_ref[...]
    raw = t * wf_ref[...] + bf_ref[...]
    col = lax.broadcasted_iota(jnp.int32, raw.shape, 1)
    tv = jnp.where(col == 0, raw, jnp.sin(raw))
    dx = x.shape[1]
    dtot = dx + tv.shape[1]
    mu = (jnp.sum(x, axis=-1, keepdims=True)
          + jnp.sum(tv, axis=-1, keepdims=True)) / dtot
    var = (jnp.sum((x - mu) ** 2, axis=-1, keepdims=True)
           + jnp.sum((tv - mu) ** 2, axis=-1, keepdims=True)) / dtot
    xh = _ln(x, mu, var, sx_ref[...], bx_ref[...])
    th = _ln(tv, mu, var, st_ref[...], bt_ref[...])
    acc = (jnp.dot(xh, wx_ref[...], preferred_element_type=jnp.float32)
           + jnp.dot(th, wt_ref[...], preferred_element_type=jnp.float32))
    o_ref[...] = _gelu(acc)


def _mm_body(x_ref, w_ref, o_ref):
    o_ref[...] = jnp.dot(x_ref[...], w_ref[...],
                         preferred_element_type=jnp.float32)


def _attn_body(q_ref, kv_ref, ew_ref, rel_ref, bm_ref, o_ref):
    q = q_ref[...]
    kv = kv_ref[...]
    k = kv[:, :_D]
    v = kv[:, _D:]
    t = q * (k + rel_ref[...])
    lb = jnp.dot(t, bm_ref[...], preferred_element_type=jnp.float32)
    lb = lb * (1.0 / np.sqrt(_DK)) + ew_ref[...]
    mx = jnp.max(lb, axis=-1, keepdims=True)
    e = jnp.exp(lb - mx)
    s = jnp.sum(e, axis=-1, keepdims=True) * (1.0 / _DK)
    o_ref[...] = (e / s) * v


def _ffn_body(h_ref, s_ref, b_ref, w1_ref, w2_ref, o_ref):
    hr = h_ref[...]
    mu = jnp.mean(hr, axis=-1, keepdims=True)
    var = jnp.mean((hr - mu) ** 2, axis=-1, keepdims=True)
    f = _ln(hr, mu, var, s_ref[...], b_ref[...])
    f = jnp.dot(_gelu(jnp.dot(f, w1_ref[...], preferred_element_type=jnp.float32)),
                w2_ref[...], preferred_element_type=jnp.float32)
    o_ref[...] = hr + f


def _ffn_msg_body(h_ref, m0_ref, m1_ref, s_ref, b_ref, w1_ref, w2_ref, o_ref):
    hr = h_ref[...] + m0_ref[...] + m1_ref[...]
    mu = jnp.mean(hr, axis=-1, keepdims=True)
    var = jnp.mean((hr - mu) ** 2, axis=-1, keepdims=True)
    f = _ln(hr, mu, var, s_ref[...], b_ref[...])
    f = jnp.dot(_gelu(jnp.dot(f, w1_ref[...], preferred_element_type=jnp.float32)),
                w2_ref[...], preferred_element_type=jnp.float32)
    o_ref[...] = hr + f


def _vote_body(raw_ref, pol_ref, src_ref, w1_ref, b1_ref, w2_ref, b2_ref,
               o_ref):
    e = jax.nn.relu(jnp.dot(raw_ref[...], w1_ref[...],
                            preferred_element_type=jnp.float32) + b1_ref[...])
    e = jnp.dot(e, w2_ref[...], preferred_element_type=jnp.float32) + b2_ref[...]
    pol = jnp.clip(pol_ref[...], 0.0, 1.0)
    o_ref[...] = src_ref[...] * e * (pol + 0.01)


def _norm_body(h_ref, s_ref, b_ref, o_ref):
    hr = h_ref[...]
    mu = jnp.mean(hr, axis=-1, keepdims=True)
    var = jnp.mean((hr - mu) ** 2, axis=-1, keepdims=True)
    o_ref[...] = jax.nn.relu(_ln(hr, mu, var, s_ref[...], b_ref[...]))


def _norm_msg_body(h_ref, m0_ref, m1_ref, s_ref, b_ref, o_ref):
    hr = h_ref[...] + m0_ref[...] + m1_ref[...]
    mu = jnp.mean(hr, axis=-1, keepdims=True)
    var = jnp.mean((hr - mu) ** 2, axis=-1, keepdims=True)
    o_ref[...] = jax.nn.relu(_ln(hr, mu, var, s_ref[...], b_ref[...]))


def _row_kernel(body, n_rows, out_cols, row_ins, full_ins):
    grid = (n_rows // _BN,)
    in_specs = ([pl.BlockSpec((_BN, a.shape[1]), lambda i: (i, 0))
                 for a in row_ins]
                + [pl.BlockSpec(a.shape, lambda i: (0, 0)) for a in full_ins])
    return pl.pallas_call(
        body,
        grid=grid,
        in_specs=in_specs,
        out_specs=pl.BlockSpec((_BN, out_cols), lambda i: (i, 0)),
        out_shape=jax.ShapeDtypeStruct((n_rows, out_cols), jnp.float32),
    )(*row_ins, *full_ins)


def _sc_gather(table, idx):
    n, dc = table.shape
    e_pad = idx.shape[0]
    bpw = e_pad // _NW
    nb = bpw // _EB
    mesh = plsc.VectorSubcoreMesh(core_axis_name="c", subcore_axis_name="s")

    @functools.partial(
        pl.kernel, mesh=mesh,
        out_type=jax.ShapeDtypeStruct((e_pad, dc), jnp.float32),
        scratch_types=[
            pltpu.VMEM((_EB,), jnp.int32),
            pltpu.VMEM((_EB, dc), jnp.float32),
            pltpu.SemaphoreType.DMA,
        ],
        compiler_params=pltpu.CompilerParams(use_tc_tiling_on_sc=False),
    )
    def k(table_hbm, idx_hbm, out_hbm, idx_v, rows_v, sem):
        wid = lax.axis_index("s") * 2 + lax.axis_index("c")
        base = wid * bpw

        def body(j, carry):
            off = base + j * _EB
            pltpu.sync_copy(idx_hbm.at[pl.ds(off, _EB)], idx_v)
            pltpu.async_copy(table_hbm.at[idx_v], rows_v, sem).wait()
            pltpu.sync_copy(rows_v, out_hbm.at[pl.ds(off, _EB)])
            return carry

        lax.fori_loop(0, nb, body, 0)

    return k(table, idx)


def _sc_scatter(m, dst, n_out):
    e_pad = m.shape[0]
    ch = min(6128, _ru(n_out, 16))
    nch = -(-n_out // ch)
    chp = ch + 16
    rpt = chp // 16
    wb = 128 if rpt % 128 == 0 else rpt
    npad = nch * ch + 16
    bpw = e_pad // _NW
    nb = bpw // _EB
    mesh = plsc.VectorSubcoreMesh(core_axis_name="c", subcore_axis_name="s")

    @functools.partial(
        pl.kernel, mesh=mesh,
        out_type=[jax.ShapeDtypeStruct((npad, _D), jnp.float32),
                  jax.ShapeDtypeStruct((npad, _D), jnp.float32)],
        scratch_types=[
            pltpu.VMEM_SHARED((chp, _D), jnp.float32),
            pltpu.VMEM((_EB,), jnp.int32),
            pltpu.VMEM((_EB,), jnp.int32),
            pltpu.VMEM((_EB, _D), jnp.float32),
            pltpu.VMEM((wb, _D), jnp.float32),
        ],
        compiler_params=pltpu.CompilerParams(use_tc_tiling_on_sc=False),
    )
    def k(m_hbm, dst_hbm, out0, out1, acc, dstv, locv, mv, sv):
        cid = lax.axis_index("c")
        sid = lax.axis_index("s")
        wid = sid * 2 + cid
        base = wid * bpw

        def zrow(r, carry):
            for c16 in range(_D // 16):
                sv[r, pl.ds(c16 * 16, 16)] = jnp.zeros((16,), jnp.float32)
            return carry

        lax.fori_loop(0, wb, zrow, 0)

        for c in range(nch):
            lo = c * ch
            for bz in range(rpt // wb):
                pltpu.sync_copy(sv, acc.at[pl.ds(sid * rpt + bz * wb, wb)])
            plsc.subcore_barrier()

            def ebody(j, carry):
                off = base + j * _EB
                pltpu.sync_copy(dst_hbm.at[pl.ds(off, _EB)], dstv)
                for i in range(_EB // 16):
                    dv = dstv[pl.ds(i * 16, 16)]
                    ok = (dv >= lo) & (dv < lo + ch)
                    locv[pl.ds(i * 16, 16)] = jnp.where(ok, dv - lo, ch)
                pltpu.sync_copy(m_hbm.at[pl.ds(off, _EB)], mv)
                pltpu.sync_copy(mv, acc.at[locv], add=True)
                return carry

            lax.fori_loop(0, nb, ebody, 0)
            plsc.subcore_barrier()

            for bw in range(rpt // wb):
                r0 = sid * rpt + bw * wb
                pltpu.sync_copy(acc.at[pl.ds(r0, wb)], mv.at[pl.ds(0, wb)])

                @pl.when(cid == 0)
                def _w0():
                    pltpu.sync_copy(mv.at[pl.ds(0, wb)],
                                    out0.at[pl.ds(lo + r0, wb)])

                @pl.when(cid == 1)
                def _w1():
                    pltpu.sync_copy(mv.at[pl.ds(0, wb)],
                                    out1.at[pl.ds(lo + r0, wb)])

    return k(m, dst)


def kernel(xs, ts, ews, edge_attr_vote, params, edge_indices):
    bm = jnp.asarray(_BM_NP)

    h = {}
    for nt in _NT:
        x = xs[nt]
        n = x.shape[0]
        np_rows = _ru(n, _BN)
        p = params['proj_' + nt]
        xp = jnp.pad(x, ((0, np_rows - n), (0, 0)))
        if nt in _TS:
            dx = x.shape[1]
            tp = jnp.pad(ts[nt].reshape(-1, 1), ((0, np_rows - n), (0, 0)))
            t2v = params['t2v']
            wf = jnp.concatenate([t2v['w0'].reshape(1), t2v['w']]).reshape(1, 12)
            bf = t2v['b'].reshape(1, 12)
            sc_ = p['ln_scale']
            bi = p['ln_bias']
            h[nt] = _row_kernel(
                _proj_ts_body, np_rows, _D, [xp, tp],
                [wf, bf, sc_[:dx].reshape(1, dx), bi[:dx].reshape(1, dx),
                 sc_[dx:].reshape(1, 12), bi[dx:].reshape(1, 12),
                 p['W'][:dx], p['W'][dx:]])
        else:
            h[nt] = _row_kernel(
                _proj_plain_body, np_rows, _D, [xp],
                [p['ln_scale'].reshape(1, -1), p['ln_bias'].reshape(1, -1),
                 p['W']])

    idx_q, idx_s, idx_d, ewp, e_pad = {}, {}, {}, {}, {}
    for et in _ET:
        kr = '__'.join(et)
        ei = edge_indices[kr]
        e = ei.shape[1]
        ep = _ru(e, _NW * _EB)
        e_pad[kr] = ep
        idx_s[kr] = jnp.pad(ei[0], (0, ep - e))
        idx_q[kr] = jnp.pad(ei[1], (0, ep - e))
        idx_d[kr] = jnp.pad(ei[1], (0, ep - e), constant_values=-1)
        ewp[kr] = jnp.pad(ews[kr], (0, ep - e)).reshape(-1, 1)

    dst_types = ('bill_version', 'subject', 'legislator_term')
    src_types = ('legislator_term', 'bill', 'donor')

    for li in range(3):
        lp = params['layer%d' % li]
        wkv = jnp.concatenate([lp['K'], lp['V']], axis=1)
        q_tab = {nt: _row_kernel(_mm_body, h[nt].shape[0], _D, [h[nt]],
                                 [lp['Q']]) for nt in dst_types}
        kv_tab = {nt: _row_kernel(_mm_body, h[nt].shape[0], 2 * _D, [h[nt]],
                                  [wkv]) for nt in src_types}
        m_et = {}
        for et in _ET:
            kr = '__'.join(et)
            src_t, dst_t = et[0], et[2]
            qe = _sc_gather(q_tab[dst_t], idx_q[kr])
            kve = _sc_gather(kv_tab[src_t], idx_s[kr])
            rel = lp['rel'][kr].reshape(1, _D)
            m_et[kr] = _row_kernel(_attn_body, e_pad[kr], _D,
                                   [qe, kve, ewp[kr]], [rel, bm])

        kr_vote = 'legislator_term__voted_on__bill_version'
        kr_hv = 'bill__has_version__bill_version'
        kr_ab = 'bill__about__subject'
        kr_dn = 'donor__donated_to__legislator_term'
        msgs = {
            'bill_version': _sc_scatter(
                jnp.concatenate([m_et[kr_vote], m_et[kr_hv]]),
                jnp.concatenate([idx_d[kr_vote], idx_d[kr_hv]]),
                h['bill_version'].shape[0]),
            'subject': _sc_scatter(m_et[kr_ab], idx_d[kr_ab],
                                   h['subject'].shape[0]),
            'legislator_term': _sc_scatter(m_et[kr_dn], idx_d[kr_dn],
                                           h['legislator_term'].shape[0]),
        }
        ffn_full = [lp['ffn_ln_scale'].reshape(1, -1),
                    lp['ffn_ln_bias'].reshape(1, -1),
                    lp['ffn_W1'], lp['ffn_W2']]
        for nt in _NT:
            nrows = h[nt].shape[0]
            if nt in msgs:
                o0, o1 = msgs[nt]
                h[nt] = _row_kernel(_ffn_msg_body, nrows, _D,
                                    [h[nt], o0[:nrows], o1[:nrows]], ffn_full)
            else:
                h[nt] = _row_kernel(_ffn_body, nrows, _D, [h[nt]], ffn_full)

    kr_v = 'legislator_term__voted_on__bill_version'
    ep = e_pad[kr_v]
    e = edge_indices[kr_v].shape[1]
    src_g = _sc_gather(h['legislator_term'], idx_s[kr_v])
    rawp = jnp.pad(edge_attr_vote[:, 1:], ((0, ep - e), (0, 0)))
    polp = jnp.pad(edge_attr_vote[:, :1], ((0, ep - e), (0, 0)))
    vp = params['vote']
    m_v = _row_kernel(_vote_body, ep, _D, [rawp, polp, src_g],
                      [vp['W1'], vp['b1'].reshape(1, -1), vp['W2'],
                       vp['b2'].reshape(1, -1)])
    v0, v1 = _sc_scatter(m_v, idx_d[kr_v], h['bill_version'].shape[0])

    outs = []
    for nt in _NT:
        nrows = h[nt].shape[0]
        pn = params['norm_' + nt]
        full = [pn['scale'].reshape(1, -1), pn['bias'].reshape(1, -1)]
        if nt == 'bill_version':
            o = _row_kernel(_norm_msg_body, nrows, _D,
                            [h[nt], v0[:nrows], v1[:nrows]], full)
        else:
            o = _row_kernel(_norm_body, nrows, _D, [h[nt]], full)
        outs.append(o[:_NN[nt]])
    return tuple(outs)
